# Initial kernel scaffold; baseline (speedup 1.0000x reference)
import jax
import jax.numpy as jnp
from jax.experimental import pallas as pl


def kernel(x, embed_table):
    raise NotImplementedError("write your pallas kernel here")


# speedup vs baseline: 3.9993x; 3.9993x over previous
import math
import functools

import jax
import jax.numpy as jnp
import numpy as np
from jax.experimental import pallas as pl
from jax.experimental.pallas import tpu as pltpu

_D_MODEL = 1024
_MAX_LEN = 2048


def _pe_table(max_len, d_model):
    pe = np.zeros((max_len, d_model), dtype=np.float32)
    pos = np.arange(max_len, dtype=np.float32)[:, None]
    div = np.exp(
        np.arange(0, d_model, 2, dtype=np.float32) * (-math.log(10000.0) / d_model)
    )
    pe[:, 0::2] = np.sin(pos * div)
    pe[:, 1::2] = np.cos(pos * div[: d_model // 2])
    return jnp.asarray(pe)


_PE = _pe_table(_MAX_LEN, _D_MODEL)

_LB = 256


def _body(x_ref, tab_ref, pe_ref, out_ref):
    row0 = tab_ref[0, :]
    diff = tab_ref[1, :] - row0
    base = pe_ref[:, :] + row0[None, :]
    w = x_ref[:, :].astype(jnp.float32)[:, :, None]
    out_ref[:, :, :] = base[None, :, :] + w * diff[None, None, :]


def kernel(x, embed_table):
    b, seq = x.shape
    d = embed_table.shape[1]
    grid = (seq // _LB,)
    return pl.pallas_call(
        _body,
        grid=grid,
        in_specs=[
            pl.BlockSpec((b, _LB), lambda i: (0, i)),
            pl.BlockSpec((2, d), lambda i: (0, 0)),
            pl.BlockSpec((_LB, d), lambda i: (i, 0)),
        ],
        out_specs=pl.BlockSpec((b, _LB, d), lambda i: (0, i, 0)),
        out_shape=jax.ShapeDtypeStruct((b, seq, d), jnp.float32),
    )(x, embed_table, _PE[:seq])
